# baseline (speedup 1.0000x reference)
import functools

import jax
import jax.numpy as jnp
from jax.experimental import pallas as pl


def _s2d(x, k):
    B, T, H, W, C = x.shape
    x = x.reshape(B, T, H // k, k, W // k, k, C)
    return x.transpose(0, 1, 2, 4, 3, 5, 6).reshape(B, T, H // k, W // k, k * k * C)


def _pad1(x):
    return jnp.pad(x, ((0, 0), (1, 1), (1, 1), (1, 1), (0, 0)))


def _w_conv1(w1):
    O = w1.shape[0]
    z = jnp.zeros((3, O), w1.dtype)
    groups = []
    for du in (0, 1):
        for dv in (0, 1):
            rows = []
            for dt in range(3):
                for a in range(4):
                    for b in range(4):
                        cols = []
                        for bh in (0, 1):
                            for bw in (0, 1):
                                dh = 4 * du + a - 3 - 2 * bh
                                dw = 4 * dv + b - 3 - 2 * bw
                                if 0 <= dh <= 2 and 0 <= dw <= 2:
                                    cols.append(w1[:, :, dt, dh, dw].T)
                                else:
                                    cols.append(z)
                        rows.append(jnp.concatenate(cols, axis=1))
            groups.append(jnp.concatenate(rows, axis=0))
    return jnp.stack(groups)


_G2 = (
    (1, 1, 0, 4, ((1, 1), (1, 2), (2, 1), (2, 2))),
    (0, 1, 2, 2, ((0, 1), (0, 2))),
    (1, 0, 1, 1, ((1, 0),)),
    (1, 0, 3, 1, ((2, 0),)),
    (0, 0, 3, 1, ((0, 0),)),
)


def _w_conv23(w):
    out = []
    for _, _, _, _, taps in _G2:
        rows = []
        for dt in range(3):
            for dh, dw in taps:
                rows.append(w[:, :, dt, dh, dw].T)
        out.append(jnp.concatenate(rows, axis=0))
    return out


def _conv1_body(x0, x1, x2, w_ref, b_ref, o_ref, *, HC, Wo):
    hb = pl.program_id(2) * HC
    xts = (x0, x1, x2)
    No = w_ref.shape[-1]
    acc = jnp.zeros((HC * Wo, No), dtype=jnp.float32)
    g = 0
    for du in (0, 1):
        for dv in (0, 1):
            xg = jnp.concatenate(
                [xt[0, 0, pl.ds(hb + du, HC), pl.ds(dv, Wo), :]
                 for xt in xts], axis=-1)
            acc = acc + jnp.dot(xg.reshape(HC * Wo, -1), w_ref[g],
                                preferred_element_type=jnp.float32)
            g += 1
    y = jnp.maximum(acc + b_ref[0][None, :], 0.0)
    o_ref[...] = y.reshape(1, 1, HC, Wo, No)


def _conv1_stage(xph, w1, b1, nh=4):
    B, Tp, Hp, Wp, Ci = xph.shape
    T, Ho, Wo = Tp - 2, Hp - 2, Wp - 2
    HC = Ho // nh
    O = w1.shape[0]
    wt = _w_conv1(w1)
    bb = jnp.tile(b1, 4).reshape(1, 4 * O)

    def xspec(dt):
        return pl.BlockSpec((1, 1, Hp, Wp, Ci),
                            lambda bi, ti, hi, dt=dt: (bi, ti + dt, 0, 0, 0))

    return pl.pallas_call(
        functools.partial(_conv1_body, HC=HC, Wo=Wo),
        grid=(B, T, nh),
        in_specs=[
            xspec(0), xspec(1), xspec(2),
            pl.BlockSpec(wt.shape, lambda bi, ti, hi: (0, 0, 0)),
            pl.BlockSpec(bb.shape, lambda bi, ti, hi: (0, 0)),
        ],
        out_specs=pl.BlockSpec((1, 1, HC, Wo, 4 * O),
                               lambda bi, ti, hi: (bi, ti, hi, 0, 0)),
        out_shape=jax.ShapeDtypeStruct((B, T, Ho, Wo, 4 * O), jnp.float32),
    )(xph, xph, xph, wt, bb)


def _conv23_body(x0, x1, x2, *rest, HC, Wo, C, relu):
    hb = pl.program_id(2) * HC
    ws = rest[:len(_G2)]
    b_ref, o_ref = rest[len(_G2)], rest[len(_G2) + 1]
    xts = (x0, x1, x2)
    No = ws[0].shape[-1]
    acc = jnp.zeros((HC * Wo, No), dtype=jnp.float32)
    for gi, (sh, sw, lo, nb, _) in enumerate(_G2):
        xg = jnp.concatenate(
            [xt[0, 0, pl.ds(hb + sh, HC), pl.ds(sw, Wo), pl.ds(lo * C, nb * C)]
             for xt in xts], axis=-1)
        acc = acc + jnp.dot(xg.reshape(HC * Wo, -1), ws[gi][...],
                            preferred_element_type=jnp.float32)
    y = acc + b_ref[0][None, :]
    if relu:
        y = jnp.maximum(y, 0.0)
    o_ref[...] = y.reshape(1, 1, HC, Wo, No)


def _conv23_stage(xph, w, b, relu, nh=1):
    B, Tp, Hp, Wp, C4 = xph.shape
    C = C4 // 4
    T, Ho, Wo = Tp - 2, Hp - 2, Wp - 2
    HC = Ho // nh
    O = w.shape[0]
    ws = _w_conv23(w)
    bb = b.reshape(1, O)

    def xspec(dt):
        return pl.BlockSpec((1, 1, Hp, Wp, C4),
                            lambda bi, ti, hi, dt=dt: (bi, ti + dt, 0, 0, 0))

    out = pl.pallas_call(
        functools.partial(_conv23_body, HC=HC, Wo=Wo, C=C, relu=relu),
        grid=(B, T, nh),
        in_specs=[xspec(0), xspec(1), xspec(2)] +
                 [pl.BlockSpec(wg.shape, lambda bi, ti, hi: (0, 0)) for wg in ws] +
                 [pl.BlockSpec(bb.shape, lambda bi, ti, hi: (0, 0))],
        out_specs=pl.BlockSpec((1, 1, HC, Wo, O),
                               lambda bi, ti, hi: (bi, ti, hi, 0, 0)),
        out_shape=jax.ShapeDtypeStruct((B, T, Ho, Wo, O), jnp.float32),
    )(xph, xph, xph, *ws, bb)
    return out


def _vq_body(f_ref, cbt_ref, emb_ref, tok_ref, e_ref):
    f = f_ref[0]
    cbt = cbt_ref[...]
    fsq = jnp.sum(f * f, axis=-1, keepdims=True)
    csq = jnp.sum(cbt * cbt, axis=0, keepdims=True)
    cross = jnp.dot(f, cbt, preferred_element_type=jnp.float32)
    dist = jnp.sqrt(jnp.maximum(fsq + csq - 2.0 * cross, 0.0))
    m = jnp.min(dist, axis=-1, keepdims=True)
    idx = jax.lax.broadcasted_iota(jnp.int32, dist.shape, 1)
    tok = jnp.min(jnp.where(dist == m, idx, jnp.int32(2 ** 30)), axis=-1)
    oh = (idx == tok[:, None]).astype(jnp.float32)
    emb = jnp.dot(oh, emb_ref[...], preferred_element_type=jnp.float32)
    tok_ref[...] = tok.reshape(tok_ref.shape)
    e_ref[...] = emb.reshape(e_ref.shape)


def _vq_stage(features, codebook, emb_table):
    B, N, D = features.shape
    K = codebook.shape[0]
    nb = 8 if N % 8 == 0 else 1
    NBLK = N // nb
    cbt = codebook.T
    tok4, emb = pl.pallas_call(
        _vq_body,
        grid=(B, nb),
        in_specs=[
            pl.BlockSpec((1, NBLK, D), lambda bi, ni: (bi, ni, 0)),
            pl.BlockSpec((D, K), lambda bi, ni: (0, 0)),
            pl.BlockSpec((K, D), lambda bi, ni: (0, 0)),
        ],
        out_specs=[
            pl.BlockSpec((1, 1, 1, NBLK), lambda bi, ni: (bi, ni, 0, 0)),
            pl.BlockSpec((1, NBLK, D), lambda bi, ni: (bi, ni, 0)),
        ],
        out_shape=[
            jax.ShapeDtypeStruct((B, nb, 1, NBLK), jnp.int32),
            jax.ShapeDtypeStruct((B, N, D), jnp.float32),
        ],
    )(features, cbt, emb_table)
    return tok4.reshape(B, N), emb


def _conv3d_xla(x, w, b, stride):
    y = jax.lax.conv_general_dilated(
        x, w, window_strides=stride,
        padding=[(1, 1), (1, 1), (1, 1)],
        dimension_numbers=("NCDHW", "OIDHW", "NCDHW"))
    return y + b[None, :, None, None, None]


def kernel(video, w1, b1, w2, b2, w3, b3, codebook, emb_table, resolution_level=0):
    del resolution_level
    f = jax.nn.relu(_conv3d_xla(video, w1, b1, (1, 2, 2)))
    f = jax.nn.relu(_conv3d_xla(f, w2, b2, (1, 2, 2)))
    f = _conv3d_xla(f, w3, b3, (1, 2, 2))
    B, C, T, H, W = f.shape
    feats = jnp.transpose(f.reshape(B, C, T * H * W), (0, 2, 1))
    tokens, emb = _vq_stage(feats, codebook, emb_table)
    return tokens, emb
